# baseline (speedup 1.0000x reference)
import functools

import jax
import jax.numpy as jnp
from jax import lax
from jax.experimental import pallas as pl
from jax.experimental.pallas import tpu as pltpu
from jax.experimental.pallas import tpu_sc as plsc

B = 4096
D = 64
N_SP, N_MV, N_IT, N_AB = 6, 24, 6, 6
N_ALL = N_SP + N_MV + N_IT + N_AB

NC, NS = 2, 16
NW = NC * NS
RW = B // NW
NBUF = 4


def _body(ids_t, sp_tab, mv_tab, it_tab, ab_tab, out,
          idx, bufs, *sems):
  wid = lax.axis_index("s") * NC + lax.axis_index("c")
  base = wid * RW
  gsems, osems = sems[:NBUF], sems[NBUF:]

  pltpu.sync_copy(ids_t.at[:, pl.ds(base, RW)], idx)

  tab_of = [sp_tab] * N_SP + [mv_tab] * N_MV + [it_tab] * N_IT \
      + [ab_tab] * N_AB

  gathers = [None] * NBUF
  for j in range(N_ALL + 1):
    if j < N_ALL:
      s = j % NBUF
      if j >= NBUF:
        pltpu.make_async_copy(out.at[pl.ds(base, RW), j], bufs.at[s],
                              osems[s]).wait()
      gathers[s] = pltpu.async_copy(
          tab_of[j].at[idx.at[j]], bufs.at[s], gsems[s])
    if j >= 1:
      jj = j - 1
      ss = jj % NBUF
      gathers[ss].wait()
      pltpu.async_copy(bufs.at[ss], out.at[pl.ds(base, RW), jj], osems[ss])

  for j in range(N_ALL - NBUF, N_ALL):
    s = j % NBUF
    pltpu.make_async_copy(out.at[pl.ds(base, RW), j], bufs.at[s],
                          osems[s]).wait()


@jax.jit
def _encode(ids_t, sp_tab, mv_tab, it_tab, ab_tab):
  mesh = plsc.VectorSubcoreMesh(core_axis_name="c", subcore_axis_name="s")
  f = pl.kernel(
      _body,
      out_type=jax.ShapeDtypeStruct((B, N_ALL, D), jnp.float32),
      mesh=mesh,
      compiler_params=pltpu.CompilerParams(use_tc_tiling_on_sc=False),
      scratch_types=[
          pltpu.VMEM((N_ALL, RW), jnp.int32),
          pltpu.VMEM((NBUF, RW, D), jnp.float32),
      ] + [pltpu.SemaphoreType.DMA] * (2 * NBUF),
  )
  out = f(ids_t, sp_tab, mv_tab, it_tab, ab_tab)
  return out.reshape(B, N_ALL * D)


def kernel(species_ids, move_ids, item_ids, ability_ids,
           species_table, move_table, item_table, ability_table):
  ids_t = jnp.concatenate(
      [species_ids.T, move_ids.T, item_ids.T, ability_ids.T],
      axis=0).astype(jnp.int32)
  return _encode(ids_t, species_table, move_table, item_table,
                 ability_table)


# speedup vs baseline: 1.1648x; 1.1648x over previous
import functools

import jax
import jax.numpy as jnp
from jax import lax
from jax.experimental import pallas as pl
from jax.experimental.pallas import tpu as pltpu
from jax.experimental.pallas import tpu_sc as plsc

B = 4096
D = 64
N_SP, N_MV, N_IT, N_AB = 6, 24, 6, 6
N_ALL = N_SP + N_MV + N_IT + N_AB
N_SMALL = N_ALL - N_SP
VSMALL = 3000

NC, NS = 2, 16
NW = NC * NS
RW = B // NW
NBUF = 4


def _body(ids_t, sp_tab, small_tabs, out, idx, sbufs, bufs, small_shared,
          *sems):
  wid = lax.axis_index("s") * NC + lax.axis_index("c")
  base = wid * RW
  ssems = sems[:N_SP]
  gsems = sems[N_SP:N_SP + NBUF]
  osems = sems[N_SP + NBUF:]

  pltpu.sync_copy(ids_t.at[:, pl.ds(base, RW)], idx)

  sp_gathers = []
  for j in range(N_SP):
    sp_gathers.append(pltpu.async_copy(
        sp_tab.at[idx.at[j]], sbufs.at[j], ssems[j]))

  @pl.when(lax.axis_index("s") == 0)
  def _():
    pltpu.sync_copy(small_tabs, small_shared)

  plsc.subcore_barrier()

  gathers = [None] * NBUF
  for t in range(N_SMALL + 1):
    if t < N_SMALL:
      j = N_SP + t
      s = t % NBUF
      if t >= NBUF:
        pltpu.make_async_copy(out.at[pl.ds(base, RW), j], bufs.at[s],
                              osems[s]).wait()
      gathers[s] = pltpu.async_copy(
          small_shared.at[idx.at[j]], bufs.at[s], gsems[s])
    if t >= 1:
      jj = N_SP + t - 1
      ss = (t - 1) % NBUF
      gathers[ss].wait()
      pltpu.async_copy(bufs.at[ss], out.at[pl.ds(base, RW), jj], osems[ss])

  for j in range(N_SP):
    sp_gathers[j].wait()
    pltpu.async_copy(sbufs.at[j], out.at[pl.ds(base, RW), j], ssems[j])

  for s in range(NBUF):
    j = N_SP + N_SMALL - NBUF + s
    pltpu.make_async_copy(out.at[pl.ds(base, RW), j], bufs.at[s],
                          osems[s]).wait()
  for j in range(N_SP):
    pltpu.make_async_copy(out.at[pl.ds(base, RW), j], sbufs.at[j],
                          ssems[j]).wait()


@jax.jit
def _encode(ids_t, sp_tab, small_tabs):
  mesh = plsc.VectorSubcoreMesh(core_axis_name="c", subcore_axis_name="s")
  f = pl.kernel(
      _body,
      out_type=jax.ShapeDtypeStruct((B, N_ALL, D), jnp.float32),
      mesh=mesh,
      compiler_params=pltpu.CompilerParams(use_tc_tiling_on_sc=False),
      scratch_types=[
          pltpu.VMEM((N_ALL, RW), jnp.int32),
          pltpu.VMEM((N_SP, RW, D), jnp.float32),
          pltpu.VMEM((NBUF, RW, D), jnp.float32),
          pltpu.VMEM_SHARED((VSMALL, D), jnp.float32),
      ] + [pltpu.SemaphoreType.DMA] * (N_SP + 2 * NBUF),
  )
  out = f(ids_t, sp_tab, small_tabs)
  return out.reshape(B, N_ALL * D)


def kernel(species_ids, move_ids, item_ids, ability_ids,
           species_table, move_table, item_table, ability_table):
  ids_t = jnp.concatenate(
      [species_ids.T.astype(jnp.int32),
       move_ids.T.astype(jnp.int32) + 0,
       item_ids.T.astype(jnp.int32) + 1000,
       ability_ids.T.astype(jnp.int32) + 2000],
      axis=0)
  small_tabs = jnp.concatenate([move_table, item_table, ability_table],
                               axis=0)
  return _encode(ids_t, species_table, small_tabs)


# speedup vs baseline: 1.9533x; 1.6770x over previous
import functools

import jax
import jax.numpy as jnp
from jax import lax
from jax.experimental import pallas as pl
from jax.experimental.pallas import tpu as pltpu
from jax.experimental.pallas import tpu_sc as plsc

B = 4096
D = 64
N_SP, N_MV, N_IT, N_AB = 6, 24, 6, 6
N_ALL = N_SP + N_MV + N_IT + N_AB
N_SMALL = N_ALL - N_SP
VSMALL = 3000

NC, NS = 2, 16
NW = NC * NS
RW = B // NW
NBUF = 4


def _body(ids_t, sp_tab, small_tabs, out, idx, sbufs, bufs, small_shared,
          *sems):
  wid = lax.axis_index("s") * NC + lax.axis_index("c")
  base = wid * RW
  ssems = sems[:N_SP]
  gsems = sems[N_SP:N_SP + NBUF]
  osems = sems[N_SP + NBUF:]

  pltpu.sync_copy(ids_t.at[:, pl.ds(base, RW)], idx)

  sp_gathers = []
  for j in range(N_SP):
    sp_gathers.append(pltpu.async_copy(
        sp_tab.at[idx.at[j]], sbufs.at[j], ssems[j]))

  @pl.when(lax.axis_index("s") == 0)
  def _():
    pltpu.sync_copy(small_tabs, small_shared)

  plsc.subcore_barrier()

  def ocol(j):
    return out.at[pl.ds(base, RW), pl.ds(j * D, D)]

  gathers = [None] * NBUF
  for t in range(N_SMALL + 1):
    if t < N_SMALL:
      j = N_SP + t
      s = t % NBUF
      if t >= NBUF:
        pltpu.make_async_copy(ocol(j), bufs.at[s], osems[s]).wait()
      gathers[s] = pltpu.async_copy(
          small_shared.at[idx.at[j]], bufs.at[s], gsems[s])
    if t >= 1:
      jj = N_SP + t - 1
      ss = (t - 1) % NBUF
      gathers[ss].wait()
      pltpu.async_copy(bufs.at[ss], ocol(jj), osems[ss])

  for j in range(N_SP):
    sp_gathers[j].wait()
    pltpu.async_copy(sbufs.at[j], ocol(j), ssems[j])

  for s in range(NBUF):
    pltpu.make_async_copy(ocol(N_SP + N_SMALL - NBUF + s), bufs.at[s],
                          osems[s]).wait()
  for j in range(N_SP):
    pltpu.make_async_copy(ocol(j), sbufs.at[j], ssems[j]).wait()


@jax.jit
def _encode(ids_t, sp_tab, small_tabs):
  mesh = plsc.VectorSubcoreMesh(core_axis_name="c", subcore_axis_name="s")
  f = pl.kernel(
      _body,
      out_type=jax.ShapeDtypeStruct((B, N_ALL * D), jnp.float32),
      mesh=mesh,
      compiler_params=pltpu.CompilerParams(use_tc_tiling_on_sc=False),
      scratch_types=[
          pltpu.VMEM((N_ALL, RW), jnp.int32),
          pltpu.VMEM((N_SP, RW, D), jnp.float32),
          pltpu.VMEM((NBUF, RW, D), jnp.float32),
          pltpu.VMEM_SHARED((VSMALL, D), jnp.float32),
      ] + [pltpu.SemaphoreType.DMA] * (N_SP + 2 * NBUF),
  )
  return f(ids_t, sp_tab, small_tabs)


def kernel(species_ids, move_ids, item_ids, ability_ids,
           species_table, move_table, item_table, ability_table):
  ids_t = jnp.concatenate(
      [species_ids.T.astype(jnp.int32),
       move_ids.T.astype(jnp.int32) + 0,
       item_ids.T.astype(jnp.int32) + 1000,
       ability_ids.T.astype(jnp.int32) + 2000],
      axis=0)
  small_tabs = jnp.concatenate([move_table, item_table, ability_table],
                               axis=0)
  return _encode(ids_t, species_table, small_tabs)


# speedup vs baseline: 2.7508x; 1.4083x over previous
import functools

import jax
import jax.numpy as jnp
from jax import lax
from jax.experimental import pallas as pl
from jax.experimental.pallas import tpu as pltpu
from jax.experimental.pallas import tpu_sc as plsc

B = 4096
D = 64
N_SP, N_MV, N_IT, N_AB = 6, 24, 6, 6
N_ALL = N_SP + N_MV + N_IT + N_AB
N_SMALL = N_ALL - N_SP
VSMALL = 3000
TILES = N_ALL * D // 128

NC, NS = 2, 16
NW = NC * NS
RW = B // NW
NBUF = 4


def _body(ids_t, sidx_t, sp_tab, small_tabs, out,
          idx, sidx, sbufs, bufs, small_shared, *sems):
  wid = lax.axis_index("s") * NC + lax.axis_index("c")
  base = wid * RW
  ssems = sems[:N_SP]
  gsems = sems[N_SP:N_SP + NBUF]
  osems = sems[N_SP + NBUF:]

  pltpu.sync_copy(ids_t.at[:, pl.ds(base, RW)], idx)
  pltpu.sync_copy(sidx_t.at[:, pl.ds(base, RW)], sidx)

  sp_gathers = []
  for j in range(N_SP):
    sp_gathers.append(pltpu.async_copy(
        sp_tab.at[idx.at[j]], sbufs.at[j], ssems[j]))

  @pl.when(lax.axis_index("s") == 0)
  def _():
    pltpu.sync_copy(small_tabs, small_shared)

  plsc.subcore_barrier()

  gathers = [None] * NBUF
  scatters = [None] * NBUF
  for t in range(N_SMALL + 1):
    if t < N_SMALL:
      j = N_SP + t
      s = t % NBUF
      if t >= NBUF:
        scatters[s].wait()
      gathers[s] = pltpu.async_copy(
          small_shared.at[idx.at[j]], bufs.at[s], gsems[s])
    if t >= 1:
      jj = N_SP + t - 1
      ss = (t - 1) % NBUF
      gathers[ss].wait()
      scatters[ss] = pltpu.async_copy(
          bufs.at[ss], out.at[sidx.at[jj]], osems[ss])

  sp_scatters = []
  for j in range(N_SP):
    sp_gathers[j].wait()
    sp_scatters.append(pltpu.async_copy(
        sbufs.at[j], out.at[sidx.at[j]], ssems[j]))

  for s in range(NBUF):
    scatters[s].wait()
  for c in sp_scatters:
    c.wait()


@jax.jit
def _encode(ids_t, sidx_t, sp_tab, small_tabs):
  mesh = plsc.VectorSubcoreMesh(core_axis_name="c", subcore_axis_name="s")
  f = pl.kernel(
      _body,
      out_type=jax.ShapeDtypeStruct((B * N_ALL, D), jnp.float32),
      mesh=mesh,
      compiler_params=pltpu.CompilerParams(use_tc_tiling_on_sc=False),
      scratch_types=[
          pltpu.VMEM((N_ALL, RW), jnp.int32),
          pltpu.VMEM((N_ALL, RW), jnp.int32),
          pltpu.VMEM((N_SP, RW, D), jnp.float32),
          pltpu.VMEM((NBUF, RW, D), jnp.float32),
          pltpu.VMEM_SHARED((VSMALL, D), jnp.float32),
      ] + [pltpu.SemaphoreType.DMA] * (N_SP + 2 * NBUF),
  )
  y = f(ids_t, sidx_t, sp_tab, small_tabs)
  y = y.reshape(B // 8, TILES, 8, 128).transpose(0, 2, 1, 3)
  return y.reshape(B, N_ALL * D)


def kernel(species_ids, move_ids, item_ids, ability_ids,
           species_table, move_table, item_table, ability_table):
  ids_t = jnp.concatenate(
      [species_ids.T.astype(jnp.int32),
       move_ids.T.astype(jnp.int32),
       item_ids.T.astype(jnp.int32) + 1000,
       ability_ids.T.astype(jnp.int32) + 2000],
      axis=0)
  small_tabs = jnp.concatenate([move_table, item_table, ability_table],
                               axis=0)
  j = jnp.arange(N_ALL, dtype=jnp.int32)[:, None]
  b = jnp.arange(B, dtype=jnp.int32)[None, :]
  sidx_t = ((b // 8) * TILES + j // 2) * 16 + (b % 8) * 2 + (j % 2)
  return _encode(ids_t, sidx_t, species_table, small_tabs)
